# baseline (speedup 1.0000x reference)
import functools

import jax
import jax.numpy as jnp
import numpy as np
from jax import lax
from jax.experimental import pallas as pl
from jax.experimental.pallas import tpu as pltpu
from jax.experimental.pallas import tpu_sc as plsc

_B, _F, _E = 16384, 26, 16
_OFFS = (np.arange(_F, dtype=np.int32) * 100000)

_NC, _NS = 2, 16
_NW = _NC * _NS
_SPW = _B // _NW
_C = 64
_NCHUNK = _SPW // _C
_RPC = _C * _F
_NIDX = _RPC // 128

_mesh = plsc.VectorSubcoreMesh(core_axis_name="c", subcore_axis_name="s")


@functools.partial(
    pl.kernel,
    mesh=_mesh,
    compiler_params=pltpu.CompilerParams(
        needs_layout_passes=False, use_tc_tiling_on_sc=False),
    out_type=[
        jax.ShapeDtypeStruct((_B, _E), jnp.float32),
        jax.ShapeDtypeStruct((_B,), jnp.float32),
    ],
    scratch_types=[
        pltpu.VMEM((_RPC,), jnp.int32),
        pltpu.VMEM((_RPC, _E), jnp.float32),
        pltpu.VMEM((_RPC,), jnp.float32),
        pltpu.VMEM((_C, _E), jnp.float32),
        pltpu.VMEM((_C,), jnp.float32),
        pltpu.SemaphoreType.DMA,
        pltpu.SemaphoreType.DMA,
    ],
)
def _sc_fm(idx_hbm, etab_hbm, ftab_hbm, cross_hbm, lin_hbm,
           idx_v, erows_v, frows_v, cross_v, lin_v, sem_e, sem_f):
    wid = lax.axis_index("s") * _NC + lax.axis_index("c")
    iot = lax.broadcasted_iota(jnp.int32, (16,), 0)
    zero16 = jnp.zeros((16,), jnp.int32)

    def chunk_body(g, carry):
        cbase = wid * _SPW + g * _C
        pltpu.sync_copy(idx_hbm.at[pl.ds(cbase * _F, _RPC)], idx_v)
        copies = []
        for j in range(_NIDX):
            copies.append(pltpu.async_copy(
                etab_hbm.at[idx_v.at[pl.ds(j * 128, 128)]],
                erows_v.at[pl.ds(j * 128, 128)], sem_e))
            copies.append(pltpu.async_copy(
                ftab_hbm.at[idx_v.at[pl.ds(j * 128, 128)]],
                frows_v.at[pl.ds(j * 128, 128)], sem_f))
        for cp in copies:
            cp.wait()

        def sample_body(b, c2):
            rbase = b * _F
            s = erows_v[rbase]
            sq = s * s
            for f in range(1, _F):
                r = erows_v[rbase + f]
                s = s + r
                sq = sq + r * r
            cross_v[b] = 0.5 * (s * s - sq)
            return c2

        lax.fori_loop(0, _C, sample_body, 0)

        for g16 in range(_C // 16):
            acc = jnp.zeros((16,), jnp.float32)
            for f in range(_F):
                idxv = iot * _F + (g16 * 16 * _F + f)
                acc = acc + plsc.load_gather(frows_v, [idxv])
            lin_v[pl.ds(g16 * 16, 16)] = acc

        pltpu.sync_copy(cross_v, cross_hbm.at[pl.ds(cbase, _C)])
        pltpu.sync_copy(lin_v, lin_hbm.at[pl.ds(cbase, _C)])
        return carry

    lax.fori_loop(0, _NCHUNK, chunk_body, 0)


_TB = 2048
_NTB = _B // _TB


def _mlp_body(cross_ref, lin_ref, w1_ref, b1_ref, w2_ref, b2_ref, w3_ref,
              c0_ref, out_ref):
    x = cross_ref[...]
    h = jnp.dot(x, w1_ref[...], preferred_element_type=jnp.float32,
                precision=lax.Precision.HIGHEST)
    h = jnp.maximum(h + b1_ref[...], 0.0)
    h = jnp.dot(h, w2_ref[...], preferred_element_type=jnp.float32,
                precision=lax.Precision.HIGHEST)
    h = jnp.maximum(h + b2_ref[...], 0.0)
    o = jnp.sum(h * w3_ref[...], axis=1)
    o = o + lin_ref[0, 0, :] + c0_ref[0, 0]
    out_ref[0, 0, :] = jax.nn.sigmoid(o)


def _mlp(cross, lin3d, w1, b1row, w2, b2row, w3row, c0):
    return pl.pallas_call(
        _mlp_body,
        grid=(_NTB,),
        in_specs=[
            pl.BlockSpec((_TB, _E), lambda i: (i, 0)),
            pl.BlockSpec((1, 1, _TB), lambda i: (i, 0, 0)),
            pl.BlockSpec((_E, 128), lambda i: (0, 0)),
            pl.BlockSpec((1, 128), lambda i: (0, 0)),
            pl.BlockSpec((128, 64), lambda i: (0, 0)),
            pl.BlockSpec((1, 64), lambda i: (0, 0)),
            pl.BlockSpec((1, 64), lambda i: (0, 0)),
            pl.BlockSpec((1, 1), lambda i: (0, 0)),
        ],
        out_specs=pl.BlockSpec((1, 1, _TB), lambda i: (i, 0, 0)),
        out_shape=jax.ShapeDtypeStruct((_NTB, 1, _TB), jnp.float32),
    )(cross, lin3d, w1, b1row, w2, b2row, w3row, c0)


def kernel(data, embed_table, fc_table, fc_bias, W1, b1, W2, b2, W3, b3):
    idx = (data.astype(jnp.int32) + jnp.asarray(_OFFS)[None, :])
    idx = idx.reshape(_B * _F)
    cross, lin = _sc_fm(idx, embed_table, fc_table.reshape(-1))
    c0 = (b3 + fc_bias).reshape(1, 1)
    out = _mlp(cross, lin.reshape(_NTB, 1, _TB), W1, b1.reshape(1, 128),
               W2, b2.reshape(1, 64), W3.reshape(1, 64), c0)
    return out.reshape(_B)
